# speedup vs baseline: 1.4929x; 1.4929x over previous
import jax
import jax.numpy as jnp
from jax.experimental import pallas as pl


_SEQ_BLK = 512


def _add_body(x_ref, pos_ref, out_ref):
    out_ref[...] = x_ref[...] + pos_ref[...]


def kernel(x, pos_emb):
    b, l, d = x.shape
    pos = pos_emb[:l]
    n_seq = l // _SEQ_BLK
    return pl.pallas_call(
        _add_body,
        grid=(n_seq, b),
        in_specs=[
            pl.BlockSpec((1, _SEQ_BLK, d), lambda s, bi: (bi, s, 0)),
            pl.BlockSpec((_SEQ_BLK, d), lambda s, bi: (s, 0)),
        ],
        out_specs=pl.BlockSpec((1, _SEQ_BLK, d), lambda s, bi: (bi, s, 0)),
        out_shape=jax.ShapeDtypeStruct((b, l, d), x.dtype),
    )(x, pos)
